# Initial kernel scaffold; baseline (speedup 1.0000x reference)
import jax
import jax.numpy as jnp
from jax.experimental import pallas as pl


def kernel(x_org, edge_index_org, batch_org, x_c1, edge_index_c1, batch_c1, x_c2, edge_index_c2, batch_c2, conv_W1, conv_b1, conv_W2, conv_b2, mlp_W, mlp_b, final_W1, final_b1, final_W2, final_b2):
    raise NotImplementedError("write your pallas kernel here")


# speedup vs baseline: 2.2876x; 2.2876x over previous
import functools

import jax
import jax.numpy as jnp
from jax import lax
from jax.experimental import pallas as pl
from jax.experimental.pallas import tpu as pltpu
from jax.experimental.pallas import tpu_sc as plsc

N = 10000
E = 320000
D = 128
H = 128
G = 64
NC_OUT = 10

NCORES = 2
NSUB = 16
NW = NCORES * NSUB
CHUNK = 128
EPW = 10112
EPAD = EPW * NW
NSTEPS = EPW // CHUNK
ACC_ROWS = 10240
ZROWS = ACC_ROWS // NSUB


@functools.partial(
    pl.kernel,
    out_type=jax.ShapeDtypeStruct((NCORES, ACC_ROWS, D), jnp.float32),
    mesh=plsc.VectorSubcoreMesh(core_axis_name="c", subcore_axis_name="s"),
    scratch_types=[
        pltpu.VMEM((CHUNK,), jnp.int32),
        pltpu.VMEM((CHUNK,), jnp.int32),
        pltpu.VMEM((CHUNK, D), jnp.float32),
        pltpu.VMEM_SHARED((ACC_ROWS, D), jnp.float32),
        pltpu.SemaphoreType.DMA,
    ],
)
def _sc_agg(x_hbm, src_hbm, dst_hbm, zeros_hbm, out_hbm,
            srcbuf, dstbuf, rows, acc, sem):
    c = lax.axis_index("c")
    s = lax.axis_index("s")

    pltpu.sync_copy(zeros_hbm, acc.at[pl.ds(s * ZROWS, ZROWS)])
    plsc.subcore_barrier()

    base = (c * NSUB + s) * EPW

    def step(i, _):
        off = base + i * CHUNK
        pltpu.sync_copy(src_hbm.at[pl.ds(off, CHUNK)], srcbuf)
        pltpu.sync_copy(dst_hbm.at[pl.ds(off, CHUNK)], dstbuf)
        pltpu.async_copy(x_hbm.at[srcbuf], rows, sem).wait()
        pltpu.sync_copy(rows, acc.at[dstbuf], add=True)
        return 0

    lax.fori_loop(0, NSTEPS, step, 0)
    plsc.subcore_barrier()

    for b in range(ZROWS // CHUNK):
        r0 = s * ZROWS + b * CHUNK
        pltpu.sync_copy(acc.at[pl.ds(r0, CHUNK)], rows)
        pltpu.sync_copy(rows, out_hbm.at[c, pl.ds(r0, CHUNK)])


BN = 2000


def _tc_layer_body(x_ref, p0_ref, p1_ref, w1_ref, b1_ref, w2_ref, b2_ref,
                   o_ref):
    h = x_ref[...] + p0_ref[0] + p1_ref[0]
    h = jnp.dot(h, w1_ref[...], preferred_element_type=jnp.float32,
                precision=lax.Precision.HIGHEST) + b1_ref[...]
    h = jnp.maximum(h, 0.0)
    h = jnp.dot(h, w2_ref[...], preferred_element_type=jnp.float32,
                precision=lax.Precision.HIGHEST) + b2_ref[...]
    o_ref[...] = jnp.maximum(h, 0.0)


def _tc_layer(x, parts, w1, b1, w2, b2):
    grid = (N // BN,)
    return pl.pallas_call(
        _tc_layer_body,
        grid=grid,
        in_specs=[
            pl.BlockSpec((BN, D), lambda i: (i, 0)),
            pl.BlockSpec((1, BN, D), lambda i: (0, i, 0)),
            pl.BlockSpec((1, BN, D), lambda i: (1, i, 0)),
            pl.BlockSpec((D, H), lambda i: (0, 0)),
            pl.BlockSpec((1, H), lambda i: (0, 0)),
            pl.BlockSpec((H, H), lambda i: (0, 0)),
            pl.BlockSpec((1, H), lambda i: (0, 0)),
        ],
        out_specs=pl.BlockSpec((BN, H), lambda i: (i, 0)),
        out_shape=jax.ShapeDtypeStruct((N, H), jnp.float32),
    )(x, parts, parts, w1, b1, w2, b2)


def _tc_layer_pool_body(x_ref, p0_ref, p1_ref, w1_ref, b1_ref, w2_ref,
                        b2_ref, batch_ref, s_ref, c_ref):
    h = x_ref[...] + p0_ref[0] + p1_ref[0]
    h = jnp.dot(h, w1_ref[...], preferred_element_type=jnp.float32,
                precision=lax.Precision.HIGHEST) + b1_ref[...]
    h = jnp.maximum(h, 0.0)
    h = jnp.dot(h, w2_ref[...], preferred_element_type=jnp.float32,
                precision=lax.Precision.HIGHEST) + b2_ref[...]
    h = jnp.maximum(h, 0.0)

    gids = lax.broadcasted_iota(jnp.int32, (BN, G), 1)
    onehot_t = (gids == batch_ref[...]).astype(jnp.float32)

    @pl.when(pl.program_id(0) == 0)
    def _():
        s_ref[...] = jnp.zeros_like(s_ref)
        c_ref[...] = jnp.zeros_like(c_ref)

    s_ref[...] += lax.dot_general(
        onehot_t, h, (((0,), (0,)), ((), ())),
        preferred_element_type=jnp.float32,
        precision=lax.Precision.HIGHEST)
    c_ref[...] += jnp.sum(onehot_t, axis=0)[None, :]


def _tc_layer_pool(x, parts, w1, b1, w2, b2, batch2):
    grid = (N // BN,)
    return pl.pallas_call(
        _tc_layer_pool_body,
        grid=grid,
        in_specs=[
            pl.BlockSpec((BN, D), lambda i: (i, 0)),
            pl.BlockSpec((1, BN, D), lambda i: (0, i, 0)),
            pl.BlockSpec((1, BN, D), lambda i: (1, i, 0)),
            pl.BlockSpec((D, H), lambda i: (0, 0)),
            pl.BlockSpec((1, H), lambda i: (0, 0)),
            pl.BlockSpec((H, H), lambda i: (0, 0)),
            pl.BlockSpec((1, H), lambda i: (0, 0)),
            pl.BlockSpec((BN, 1), lambda i: (i, 0)),
        ],
        out_specs=[
            pl.BlockSpec((G, H), lambda i: (0, 0)),
            pl.BlockSpec((1, G), lambda i: (0, 0)),
        ],
        out_shape=[
            jax.ShapeDtypeStruct((G, H), jnp.float32),
            jax.ShapeDtypeStruct((1, G), jnp.float32),
        ],
    )(x, parts, parts, w1, b1, w2, b2, batch2)


def _tc_head_body(s0_ref, c0_ref, s1_ref, c1_ref, s2_ref, c2_ref,
                  mw_ref, mb_ref, fw1_ref, fb1_ref, fw2_ref, fb2_ref,
                  o_ref):
    def hg(s_ref, c_ref):
        cnt = jnp.maximum(c_ref[...], 1.0)
        pooled = s_ref[...] / cnt.reshape(G, 1)
        return jnp.dot(pooled, mw_ref[...],
                       preferred_element_type=jnp.float32,
                       precision=lax.Precision.HIGHEST) + mb_ref[...]

    h0 = hg(s0_ref, c0_ref)
    h1 = hg(s1_ref, c1_ref)
    h2 = hg(s2_ref, c2_ref)
    acc = (jnp.dot(h0, fw1_ref[0:H, :], preferred_element_type=jnp.float32,
                   precision=lax.Precision.HIGHEST)
           + jnp.dot(h1, fw1_ref[H:2 * H, :],
                     preferred_element_type=jnp.float32,
                     precision=lax.Precision.HIGHEST)
           + jnp.dot(h2, fw1_ref[2 * H:3 * H, :],
                     preferred_element_type=jnp.float32,
                     precision=lax.Precision.HIGHEST))
    acc = jnp.maximum(acc + fb1_ref[...], 0.0)
    o_ref[...] = jnp.dot(acc, fw2_ref[...], preferred_element_type=jnp.float32,
                         precision=lax.Precision.HIGHEST) + fb2_ref[...]


def _tc_head(s0, c0, s1, c1, s2, c2, mlp_W, mlp_b2, fW1, fb1_2, fW2, fb2_2):
    return pl.pallas_call(
        _tc_head_body,
        out_shape=jax.ShapeDtypeStruct((G, NC_OUT), jnp.float32),
    )(s0, c0, s1, c1, s2, c2, mlp_W, mlp_b2, fW1, fb1_2, fW2, fb2_2)


def kernel(x_org, edge_index_org, batch_org, x_c1, edge_index_c1, batch_c1,
           x_c2, edge_index_c2, batch_c2, conv_W1, conv_b1, conv_W2, conv_b2,
           mlp_W, mlp_b, final_W1, final_b1, final_W2, final_b2):
    zeros = jnp.zeros((ZROWS, D), jnp.float32)
    pad_src = jnp.zeros((EPAD - E,), jnp.int32)
    pad_dst = jnp.full((EPAD - E,), N, jnp.int32)

    b1r = conv_b1.reshape(3, 1, H)
    b2r = conv_b2.reshape(3, 1, H)

    def branch(x, ei, batch):
        src = jnp.concatenate([ei[0], pad_src])
        dst = jnp.concatenate([ei[1], pad_dst])
        batch2 = batch.reshape(N, 1)
        for l in range(2):
            parts = _sc_agg(x, src, dst, zeros)
            x = _tc_layer(x, parts, conv_W1[l], b1r[l], conv_W2[l], b2r[l])
        parts = _sc_agg(x, src, dst, zeros)
        return _tc_layer_pool(x, parts, conv_W1[2], b1r[2], conv_W2[2],
                              b2r[2], batch2)

    s0, c0 = branch(x_org, edge_index_org, batch_org)
    s1, c1 = branch(x_c1, edge_index_c1, batch_c1)
    s2, c2 = branch(x_c2, edge_index_c2, batch_c2)

    return _tc_head(s0, c0, s1, c1, s2, c2,
                    mlp_W, mlp_b.reshape(1, H),
                    final_W1, final_b1.reshape(1, H),
                    final_W2, final_b2.reshape(1, NC_OUT))
